# baseline (speedup 1.0000x reference)
import functools

import jax
import jax.numpy as jnp
from jax import lax
from jax.experimental import pallas as pl
from jax.experimental.pallas import tpu as pltpu
from jax.experimental.pallas import tpu_sc as plsc

_N = 10000
_NP = 10240
_E = 320000
_B = 64
_ROWS = _E // 128
_NSC = 2
_NTILE = 16
_RA = 157
_RB = 79
_SSEG = _B * _NP // _NTILE
_DSEG = _NP // _NTILE
_NSELF = (_N + 127) // 128
_NBPAD = _NSELF * 128


def _sc_build(edge_flat, batch):
    mesh = plsc.VectorSubcoreMesh(core_axis_name="c", subcore_axis_name="s")

    @functools.partial(
        pl.kernel,
        mesh=mesh,
        compiler_params=pltpu.CompilerParams(needs_layout_passes=False),
        out_type=[
            jax.ShapeDtypeStruct((_NSC * _B * _NP,), jnp.float32),
            jax.ShapeDtypeStruct((_B,), jnp.float32),
        ],
        scratch_types=[
            pltpu.VMEM((_RA * 128,), jnp.int32),
            pltpu.VMEM((_RB * 128,), jnp.int32),
            pltpu.VMEM((_RB * 128,), jnp.int32),
            pltpu.VMEM((_NBPAD,), jnp.int32),
            pltpu.VMEM((_NP,), jnp.float32),
            pltpu.VMEM((_NP,), jnp.float32),
            pltpu.VMEM((4 * _DSEG,), jnp.float32),
            pltpu.VMEM((_B,), jnp.float32),
            pltpu.VMEM((_B,), jnp.int32),
        ] + [pltpu.VMEM((128,), jnp.int32) for _ in range(8)]
          + [pltpu.VMEM((128,), jnp.float32) for _ in range(8)]
          + [pltpu.SemaphoreType.DMA for _ in range(10)]
          + [
            pltpu.VMEM_SHARED((_NP,), jnp.float32),
            pltpu.VMEM_SHARED((_NTILE, _NP), jnp.float32),
            pltpu.VMEM_SHARED((_B,), jnp.float32),
            pltpu.VMEM_SHARED((_B * _NP,), jnp.float32),
        ],
    )
    def sc(edge_h, batch_h, s_out, cnt_out,
           dstA, srcB, dstB, batch_v, dis_v, histv, tmpred, cntv, idc,
           ib0, ib1, ib2, ib3, ib4, ib5, ib6, ib7,
           vb0, vb1, vb2, vb3, vb4, vb5, vb6, vb7,
           sm0, sm1, sm2, sm3, sm4, sm5, sm6, sm7, sm8, sm9,
           dis_sh, hist_sh, cnt_sh, s_sh):
        idx_bufs = (ib0, ib1, ib2, ib3, ib4, ib5, ib6, ib7)
        val_bufs = (vb0, vb1, vb2, vb3, vb4, vb5, vb6, vb7)
        sems = (sm0, sm1, sm2, sm3, sm6, sm7, sm8, sm9)
        c = lax.axis_index("c")
        s = lax.axis_index("s")
        w = s * _NSC + c

        base_a = (s * _ROWS) // _NTILE
        end_a = ((s + 1) * _ROWS) // _NTILE
        base_b = (w * _ROWS) // (_NSC * _NTILE)
        end_b = ((w + 1) * _ROWS) // (_NSC * _NTILE)

        pltpu.async_copy(
            edge_h.at[1, pl.ds(base_a * 128, _RA * 128)], dstA, sm4)
        pltpu.async_copy(batch_h, batch_v.at[pl.ds(0, _N)], sm4)
        pltpu.async_copy(
            edge_h.at[0, pl.ds(base_b * 128, _RB * 128)], srcB, sm5)
        pltpu.async_copy(
            edge_h.at[1, pl.ds(base_b * 128, _RB * 128)], dstB, sm5)

        iota16 = lax.iota(jnp.int32, 16)
        zf = jnp.zeros((16,), jnp.float32)
        zi = jnp.zeros((16,), jnp.int32)

        def _zero(i, carry):
            dis_v[pl.ds(i * 16, 16)] = zf
            histv[pl.ds(i * 16, 16)] = zf
            return carry
        lax.fori_loop(0, _NP // 16, _zero, 0)
        for t in range((_NBPAD - _N) // 16):
            batch_v[pl.ds(_N + t * 16, 16)] = zi
        for t in range(_B // 16):
            cntv[pl.ds(t * 16, 16)] = zf
            idc[pl.ds(t * 16, 16)] = iota16 + 16 * t

        for t in range(_SSEG // _NP):
            pltpu.async_copy(dis_v, s_sh.at[pl.ds(s * _SSEG + t * _NP, _NP)],
                             sems[t])

        @pl.when(s == 0)
        def _():
            pltpu.sync_copy(dis_v.at[pl.ds(0, _B)], cnt_sh)

        pltpu.make_async_copy(
            edge_h.at[1, pl.ds(base_a * 128, _RA * 128)], dstA, sm4).wait()
        pltpu.make_async_copy(batch_h, batch_v.at[pl.ds(0, _N)], sm4).wait()

        ones16 = jnp.ones((16,), jnp.float32)

        def _phase_a(i, carry):
            dd = dstA[pl.ds(i * 16, 16)]
            plsc.addupdate_scatter(histv, [dd], ones16)
            return carry
        lax.fori_loop(0, (end_a - base_a) * 8, _phase_a, 0)

        pltpu.sync_copy(histv, hist_sh.at[s])
        plsc.subcore_barrier()
        for p in range(_NTILE // 4):
            for r4 in range(4):
                r = p * 4 + r4
                pltpu.async_copy(
                    hist_sh.at[r, pl.ds(s * _DSEG, _DSEG)],
                    tmpred.at[pl.ds(r4 * _DSEG, _DSEG)], sm4)
            for r4 in range(4):
                r = p * 4 + r4
                pltpu.make_async_copy(
                    hist_sh.at[r, pl.ds(s * _DSEG, _DSEG)],
                    tmpred.at[pl.ds(r4 * _DSEG, _DSEG)], sm4).wait()

            def _reduce(i, carry, p=p):
                acc = tmpred[pl.ds(i * 16, 16)]
                for r4 in range(1, 4):
                    acc = acc + tmpred[pl.ds(r4 * _DSEG + i * 16, 16)]
                if p > 0:
                    acc = acc + histv[pl.ds(i * 16, 16)]
                histv[pl.ds(i * 16, 16)] = acc
                return carry
            lax.fori_loop(0, _DSEG // 16, _reduce, 0)

        def _mkdis(i, carry):
            d = histv[pl.ds(i * 16, 16)] + 1.0
            seed = plsc.bitcast(
                jnp.int32(0x5F3759DF) - lax.shift_right_logical(
                    plsc.bitcast(d, jnp.int32), 1),
                jnp.float32)
            hd = 0.5 * d
            y = seed
            for _ in range(3):
                y = y * (1.5 - hd * y * y)
            histv[pl.ds(i * 16, 16)] = y
            return carry
        lax.fori_loop(0, _DSEG // 16, _mkdis, 0)
        pltpu.sync_copy(histv.at[pl.ds(0, _DSEG)],
                        dis_sh.at[pl.ds(s * _DSEG, _DSEG)])

        for t in range(_SSEG // _NP):
            pltpu.make_async_copy(
                dis_v, s_sh.at[pl.ds(s * _SSEG + t * _NP, _NP)],
                sems[t]).wait()

        plsc.subcore_barrier()
        pltpu.sync_copy(dis_sh, dis_v)

        for j in range(3):
            cch = w + _NSC * _NTILE * j

            @pl.when(cch < _NSELF)
            def _(cch=cch):
                for k in range(8):
                    off = cch * 128 + k * 16
                    nn = off + iota16
                    bb = batch_v[pl.ds(off, 16)]
                    fs = dis_v[pl.ds(off, 16)]
                    idx_bufs[j][pl.ds(k * 16, 16)] = bb * _NP + nn
                    val_bufs[j][pl.ds(k * 16, 16)] = jnp.where(
                        nn < _N, fs * fs, 0.0)
                pltpu.async_copy(val_bufs[j], s_sh.at[idx_bufs[j]],
                                 sems[j], add=True)

        for j in range(3):
            cch = w + _NSC * _NTILE * j

            @pl.when(cch < _NSELF)
            def _(j=j, cch=cch):
                pltpu.make_async_copy(
                    val_bufs[j], s_sh.at[idx_bufs[j]], sems[j]).wait()

        for j in range(5):
            cch = s + _NTILE * j

            @pl.when(cch < _NSELF)
            def _(cch=cch):
                for k in range(8):
                    off = cch * 128 + k * 16
                    bb = batch_v[pl.ds(off, 16)]
                    ok = (off + iota16) < _N
                    plsc.addupdate_scatter(
                        cntv, [bb], jnp.where(ok, 1.0, 0.0))
        pltpu.sync_copy(cntv, cnt_sh.at[idc], add=True)

        pltpu.make_async_copy(
            edge_h.at[0, pl.ds(base_b * 128, _RB * 128)], srcB, sm5).wait()
        pltpu.make_async_copy(
            edge_h.at[1, pl.ds(base_b * 128, _RB * 128)], dstB, sm5).wait()

        n_b = end_b - base_b

        def _phase_b(g, carry):
            for b in range(8):
                j = g * 8 + b

                @pl.when(j < n_b)
                def _(j=j, b=b):
                    @pl.when(j >= 8)
                    def _():
                        pltpu.make_async_copy(
                            val_bufs[b], s_sh.at[idx_bufs[b]], sems[b]).wait()
                    for k in range(8):
                        o = j * 128 + k * 16
                        dd = dstB[pl.ds(o, 16)]
                        ss = srcB[pl.ds(o, 16)]
                        bb = plsc.load_gather(batch_v, [dd])
                        fd = plsc.load_gather(dis_v, [dd])
                        fs = plsc.load_gather(dis_v, [ss])
                        idx_bufs[b][pl.ds(k * 16, 16)] = bb * _NP + ss
                        val_bufs[b][pl.ds(k * 16, 16)] = fd * fs
                    pltpu.async_copy(val_bufs[b], s_sh.at[idx_bufs[b]],
                                     sems[b], add=True)
            return carry
        lax.fori_loop(0, (_RB + 7) // 8, _phase_b, 0)
        for b in range(8):
            pltpu.make_async_copy(
                val_bufs[b], s_sh.at[idx_bufs[b]], sems[b]).wait()

        plsc.subcore_barrier()

        for t in range(_SSEG // _NP):
            pltpu.async_copy(
                s_sh.at[pl.ds(s * _SSEG + t * _NP, _NP)],
                s_out.at[pl.ds(c * (_B * _NP) + s * _SSEG + t * _NP, _NP)],
                sems[t])
        for t in range(_SSEG // _NP):
            pltpu.make_async_copy(
                s_sh.at[pl.ds(s * _SSEG + t * _NP, _NP)],
                s_out.at[pl.ds(c * (_B * _NP) + s * _SSEG + t * _NP, _NP)],
                sems[t]).wait()

        @pl.when(jnp.logical_and(c == 0, s == 0))
        def _():
            pltpu.sync_copy(cnt_sh, cnt_out)

    return sc(edge_flat, batch)


_KBLK = 1024
_KSTEPS = _NP // _KBLK


def _tc_dense(s_slabs, x_p, cnt2, wc, bc, w1, b1, w2, b2, w3, b3):
    def body(s_ref, x_ref, cnt_ref, wc_ref, bc_ref, w1_ref, b1_ref,
             w2_ref, b2_ref, w3_ref, b3_ref, o_ref, acc):
        k = pl.program_id(0)

        @pl.when(k == 0)
        def _():
            acc[...] = jnp.zeros_like(acc)

        sv = s_ref[...]
        acc[...] += jnp.dot(sv[0] + sv[1], x_ref[...],
                            preferred_element_type=jnp.float32,
                            precision=lax.Precision.HIGHEST)

        @pl.when(k == _KSTEPS - 1)
        def _():
            dot = functools.partial(jnp.dot,
                                    preferred_element_type=jnp.float32,
                                    precision=lax.Precision.HIGHEST)
            pooled = dot(acc[...], wc_ref[...]) + cnt_ref[...] * bc_ref[...]
            h1 = jnp.maximum(dot(pooled, w1_ref[...]) + b1_ref[...], 0.0)
            h2 = jnp.maximum(dot(h1, w2_ref[...]) + b2_ref[...], 0.0)
            lg = dot(h2, w3_ref[...]) + b3_ref[...]
            mx = jnp.max(lg, axis=1, keepdims=True)
            ex = jnp.exp(lg - mx)
            o_ref[...] = ex / jnp.sum(ex, axis=1, keepdims=True)

    full = lambda a: pl.BlockSpec(a.shape, lambda k: (0,) * a.ndim)
    return pl.pallas_call(
        body,
        grid=(_KSTEPS,),
        in_specs=[
            pl.BlockSpec((2, _B, _KBLK), lambda k: (0, 0, k)),
            pl.BlockSpec((_KBLK, 128), lambda k: (k, 0)),
            full(cnt2),
            full(wc), full(bc), full(w1), full(b1),
            full(w2), full(b2), full(w3), full(b3),
        ],
        out_specs=pl.BlockSpec((_B, 2), lambda k: (0, 0)),
        out_shape=jax.ShapeDtypeStruct((_B, 2), jnp.float32),
        scratch_shapes=[
            pltpu.VMEM((_B, 128), jnp.float32),
        ],
    )(s_slabs, x_p, cnt2, wc, bc, w1, b1, w2, b2, w3, b3)


def kernel(x, edge_index, batch, W_conv, b_conv, W1, b1, W2, b2, W3, b3):
    edge_i = jnp.asarray(edge_index, jnp.int32)
    batch_i = jnp.asarray(batch, jnp.int32)
    s_flat, cnt = _sc_build(edge_i, batch_i)
    s_slabs = s_flat.reshape(_NSC, _B, _NP)
    x_p = jnp.pad(x, ((0, _NP - _N), (0, 0)))
    return _tc_dense(s_slabs, x_p, cnt.reshape(_B, 1),
                     W_conv, b_conv.reshape(1, -1), W1, b1.reshape(1, -1),
                     W2, b2.reshape(1, -1), W3, b3.reshape(1, -1))


# speedup vs baseline: 1.0361x; 1.0361x over previous
import functools

import jax
import jax.numpy as jnp
from jax import lax
from jax.experimental import pallas as pl
from jax.experimental.pallas import tpu as pltpu
from jax.experimental.pallas import tpu_sc as plsc

_N = 10000
_NP = 10240
_E = 320000
_B = 64
_ROWS = _E // 128
_NSC = 2
_NTILE = 16
_RA = 157
_RB = 79
_SSEG = _B * _NP // _NTILE
_DSEG = _NP // _NTILE
_NSELF = (_N + 127) // 128
_NBPAD = _NSELF * 128


def _sc_build(edge_flat, batch):
    mesh = plsc.VectorSubcoreMesh(core_axis_name="c", subcore_axis_name="s")

    @functools.partial(
        pl.kernel,
        mesh=mesh,
        compiler_params=pltpu.CompilerParams(needs_layout_passes=False),
        out_type=[
            jax.ShapeDtypeStruct((_NSC * _B * _NP,), jnp.float32),
            jax.ShapeDtypeStruct((_B,), jnp.float32),
        ],
        scratch_types=[
            pltpu.VMEM((_RA * 128,), jnp.int32),
            pltpu.VMEM((_RB * 128,), jnp.int32),
            pltpu.VMEM((_RB * 128,), jnp.int32),
            pltpu.VMEM((_NBPAD,), jnp.int32),
            pltpu.VMEM((_NP,), jnp.float32),
            pltpu.VMEM((_NP,), jnp.float32),
            pltpu.VMEM((4 * _DSEG,), jnp.float32),
            pltpu.VMEM((_B,), jnp.float32),
            pltpu.VMEM((_B,), jnp.int32),
        ] + [pltpu.VMEM((128,), jnp.int32) for _ in range(8)]
          + [pltpu.VMEM((128,), jnp.float32) for _ in range(8)]
          + [pltpu.SemaphoreType.DMA for _ in range(10)]
          + [
            pltpu.VMEM_SHARED((_NP,), jnp.float32),
            pltpu.VMEM_SHARED((_NTILE, _NP), jnp.float32),
            pltpu.VMEM_SHARED((_B,), jnp.float32),
            pltpu.VMEM_SHARED((_B * _NP,), jnp.float32),
        ],
    )
    def sc(edge_h, batch_h, s_out, cnt_out,
           dstA, srcB, dstB, batch_v, dis_v, histv, tmpred, cntv, idc,
           ib0, ib1, ib2, ib3, ib4, ib5, ib6, ib7,
           vb0, vb1, vb2, vb3, vb4, vb5, vb6, vb7,
           sm0, sm1, sm2, sm3, sm4, sm5, sm6, sm7, sm8, sm9,
           dis_sh, hist_sh, cnt_sh, s_sh):
        idx_bufs = (ib0, ib1, ib2, ib3, ib4, ib5, ib6, ib7)
        val_bufs = (vb0, vb1, vb2, vb3, vb4, vb5, vb6, vb7)
        sems = (sm0, sm1, sm2, sm3, sm6, sm7, sm8, sm9)
        c = lax.axis_index("c")
        s = lax.axis_index("s")
        w = s * _NSC + c

        base_a = (s * _ROWS) // _NTILE
        end_a = ((s + 1) * _ROWS) // _NTILE
        base_b = (w * _ROWS) // (_NSC * _NTILE)
        end_b = ((w + 1) * _ROWS) // (_NSC * _NTILE)

        pltpu.async_copy(
            edge_h.at[1, pl.ds(base_a * 128, _RA * 128)], dstA, sm4)
        pltpu.async_copy(batch_h, batch_v.at[pl.ds(0, _N)], sm4)
        pltpu.async_copy(
            edge_h.at[0, pl.ds(base_b * 128, _RB * 128)], srcB, sm5)
        pltpu.async_copy(
            edge_h.at[1, pl.ds(base_b * 128, _RB * 128)], dstB, sm5)

        iota16 = lax.iota(jnp.int32, 16)
        zf = jnp.zeros((16,), jnp.float32)
        zi = jnp.zeros((16,), jnp.int32)

        def _zero(i, carry):
            dis_v[pl.ds(i * 16, 16)] = zf
            histv[pl.ds(i * 16, 16)] = zf
            return carry
        lax.fori_loop(0, _NP // 16, _zero, 0)
        for t in range((_NBPAD - _N) // 16):
            batch_v[pl.ds(_N + t * 16, 16)] = zi
        for t in range(_B // 16):
            cntv[pl.ds(t * 16, 16)] = zf
            idc[pl.ds(t * 16, 16)] = iota16 + 16 * t

        for t in range(_SSEG // _NP):
            pltpu.async_copy(dis_v, s_sh.at[pl.ds(s * _SSEG + t * _NP, _NP)],
                             sems[t])

        @pl.when(s == 0)
        def _():
            pltpu.sync_copy(dis_v.at[pl.ds(0, _B)], cnt_sh)

        pltpu.make_async_copy(
            edge_h.at[1, pl.ds(base_a * 128, _RA * 128)], dstA, sm4).wait()
        pltpu.make_async_copy(batch_h, batch_v.at[pl.ds(0, _N)], sm4).wait()

        ones16 = jnp.ones((16,), jnp.float32)

        def _phase_a(i, carry):
            dd = dstA[pl.ds(i * 16, 16)]
            plsc.addupdate_scatter(histv, [dd], ones16)
            return carry
        lax.fori_loop(0, (end_a - base_a) * 8, _phase_a, 0)

        pltpu.sync_copy(histv, hist_sh.at[s])
        plsc.subcore_barrier()
        for p in range(_NTILE // 4):
            for r4 in range(4):
                r = p * 4 + r4
                pltpu.async_copy(
                    hist_sh.at[r, pl.ds(s * _DSEG, _DSEG)],
                    tmpred.at[pl.ds(r4 * _DSEG, _DSEG)], sm4)
            for r4 in range(4):
                r = p * 4 + r4
                pltpu.make_async_copy(
                    hist_sh.at[r, pl.ds(s * _DSEG, _DSEG)],
                    tmpred.at[pl.ds(r4 * _DSEG, _DSEG)], sm4).wait()

            def _reduce(i, carry, p=p):
                acc = tmpred[pl.ds(i * 16, 16)]
                for r4 in range(1, 4):
                    acc = acc + tmpred[pl.ds(r4 * _DSEG + i * 16, 16)]
                if p > 0:
                    acc = acc + histv[pl.ds(i * 16, 16)]
                histv[pl.ds(i * 16, 16)] = acc
                return carry
            lax.fori_loop(0, _DSEG // 16, _reduce, 0)

        def _mkdis(i, carry):
            d = histv[pl.ds(i * 16, 16)] + 1.0
            seed = plsc.bitcast(
                jnp.int32(0x5F3759DF) - lax.shift_right_logical(
                    plsc.bitcast(d, jnp.int32), 1),
                jnp.float32)
            hd = 0.5 * d
            y = seed
            for _ in range(3):
                y = y * (1.5 - hd * y * y)
            histv[pl.ds(i * 16, 16)] = y
            return carry
        lax.fori_loop(0, _DSEG // 16, _mkdis, 0)
        pltpu.sync_copy(histv.at[pl.ds(0, _DSEG)],
                        dis_sh.at[pl.ds(s * _DSEG, _DSEG)])

        for t in range(_SSEG // _NP):
            pltpu.make_async_copy(
                dis_v, s_sh.at[pl.ds(s * _SSEG + t * _NP, _NP)],
                sems[t]).wait()

        plsc.subcore_barrier()
        pltpu.sync_copy(dis_sh, dis_v)

        for j in range(3):
            cch = w + _NSC * _NTILE * j

            @pl.when(cch < _NSELF)
            def _(cch=cch):
                for k in range(8):
                    off = cch * 128 + k * 16
                    nn = off + iota16
                    bb = batch_v[pl.ds(off, 16)]
                    fs = dis_v[pl.ds(off, 16)]
                    idx_bufs[j][pl.ds(k * 16, 16)] = bb * _NP + nn
                    val_bufs[j][pl.ds(k * 16, 16)] = jnp.where(
                        nn < _N, fs * fs, 0.0)
                pltpu.async_copy(val_bufs[j], s_sh.at[idx_bufs[j]],
                                 sems[j], add=True)

        for j in range(3):
            cch = w + _NSC * _NTILE * j

            @pl.when(cch < _NSELF)
            def _(j=j, cch=cch):
                pltpu.make_async_copy(
                    val_bufs[j], s_sh.at[idx_bufs[j]], sems[j]).wait()

        for j in range(5):
            cch = s + _NTILE * j

            @pl.when(cch < _NSELF)
            def _(cch=cch):
                for k in range(8):
                    off = cch * 128 + k * 16
                    bb = batch_v[pl.ds(off, 16)]
                    ok = (off + iota16) < _N
                    plsc.addupdate_scatter(
                        cntv, [bb], jnp.where(ok, 1.0, 0.0))
        pltpu.sync_copy(cntv, cnt_sh.at[idc], add=True)

        pltpu.make_async_copy(
            edge_h.at[0, pl.ds(base_b * 128, _RB * 128)], srcB, sm5).wait()
        pltpu.make_async_copy(
            edge_h.at[1, pl.ds(base_b * 128, _RB * 128)], dstB, sm5).wait()

        n_b = end_b - base_b

        def _phase_b(g, carry):
            for b in range(8):
                j = g * 8 + b

                @pl.when(j < n_b)
                def _(j=j, b=b):
                    @pl.when(j >= 8)
                    def _():
                        pltpu.make_async_copy(
                            val_bufs[b], s_sh.at[idx_bufs[b]], sems[b]).wait()
                    for k in range(8):
                        o = j * 128 + k * 16
                        dd = dstB[pl.ds(o, 16)]
                        ss = srcB[pl.ds(o, 16)]
                        bb = plsc.load_gather(batch_v, [dd])
                        fd = plsc.load_gather(dis_v, [dd])
                        fs = plsc.load_gather(dis_v, [ss])
                        idx_bufs[b][pl.ds(k * 16, 16)] = bb * _NP + ss
                        val_bufs[b][pl.ds(k * 16, 16)] = fd * fs
                    pltpu.async_copy(val_bufs[b], s_sh.at[idx_bufs[b]],
                                     sems[b], add=True)
            return carry
        lax.fori_loop(0, (_RB + 7) // 8, _phase_b, 0)
        for b in range(8):
            pltpu.make_async_copy(
                val_bufs[b], s_sh.at[idx_bufs[b]], sems[b]).wait()

        plsc.subcore_barrier()

        for t in range(_SSEG // _NP):
            pltpu.async_copy(
                s_sh.at[pl.ds(s * _SSEG + t * _NP, _NP)],
                s_out.at[pl.ds(c * (_B * _NP) + s * _SSEG + t * _NP, _NP)],
                sems[t])
        for t in range(_SSEG // _NP):
            pltpu.make_async_copy(
                s_sh.at[pl.ds(s * _SSEG + t * _NP, _NP)],
                s_out.at[pl.ds(c * (_B * _NP) + s * _SSEG + t * _NP, _NP)],
                sems[t]).wait()

        @pl.when(jnp.logical_and(c == 0, s == 0))
        def _():
            pltpu.sync_copy(cnt_sh, cnt_out)

    return sc(edge_flat, batch)


_KBLK = 2048
_KSTEPS = _NP // _KBLK


def _tc_dense(s_slabs, x_p, cnt2, wc, bc, w1, b1, w2, b2, w3, b3):
    def body(s_ref, x_ref, cnt_ref, wc_ref, bc_ref, w1_ref, b1_ref,
             w2_ref, b2_ref, w3_ref, b3_ref, o_ref, acc):
        k = pl.program_id(0)

        @pl.when(k == 0)
        def _():
            acc[...] = jnp.zeros_like(acc)

        sv = s_ref[...]
        acc[...] += jnp.dot(sv[0] + sv[1], x_ref[...],
                            preferred_element_type=jnp.float32,
                            precision=lax.Precision.HIGHEST)

        @pl.when(k == _KSTEPS - 1)
        def _():
            dot = functools.partial(jnp.dot,
                                    preferred_element_type=jnp.float32,
                                    precision=lax.Precision.HIGHEST)
            pooled = dot(acc[...], wc_ref[...]) + cnt_ref[...] * bc_ref[...]
            h1 = jnp.maximum(dot(pooled, w1_ref[...]) + b1_ref[...], 0.0)
            h2 = jnp.maximum(dot(h1, w2_ref[...]) + b2_ref[...], 0.0)
            lg = dot(h2, w3_ref[...]) + b3_ref[...]
            mx = jnp.max(lg, axis=1, keepdims=True)
            ex = jnp.exp(lg - mx)
            o_ref[...] = ex / jnp.sum(ex, axis=1, keepdims=True)

    full = lambda a: pl.BlockSpec(a.shape, lambda k: (0,) * a.ndim)
    return pl.pallas_call(
        body,
        grid=(_KSTEPS,),
        in_specs=[
            pl.BlockSpec((2, _B, _KBLK), lambda k: (0, 0, k)),
            pl.BlockSpec((_KBLK, 128), lambda k: (k, 0)),
            full(cnt2),
            full(wc), full(bc), full(w1), full(b1),
            full(w2), full(b2), full(w3), full(b3),
        ],
        out_specs=pl.BlockSpec((_B, 2), lambda k: (0, 0)),
        out_shape=jax.ShapeDtypeStruct((_B, 2), jnp.float32),
        scratch_shapes=[
            pltpu.VMEM((_B, 128), jnp.float32),
        ],
    )(s_slabs, x_p, cnt2, wc, bc, w1, b1, w2, b2, w3, b3)


def kernel(x, edge_index, batch, W_conv, b_conv, W1, b1, W2, b2, W3, b3):
    edge_i = jnp.asarray(edge_index, jnp.int32)
    batch_i = jnp.asarray(batch, jnp.int32)
    s_flat, cnt = _sc_build(edge_i, batch_i)
    s_slabs = s_flat.reshape(_NSC, _B, _NP)
    x_p = jnp.pad(x, ((0, _NP - _N), (0, 0)))
    return _tc_dense(s_slabs, x_p, cnt.reshape(_B, 1),
                     W_conv, b_conv.reshape(1, -1), W1, b1.reshape(1, -1),
                     W2, b2.reshape(1, -1), W3, b3.reshape(1, -1))


# speedup vs baseline: 1.1851x; 1.1438x over previous
import functools

import jax
import jax.numpy as jnp
from jax import lax
from jax.experimental import pallas as pl
from jax.experimental.pallas import tpu as pltpu
from jax.experimental.pallas import tpu_sc as plsc

_N = 10000
_NP = 10240
_E = 320000
_B = 64
_ROWS = _E // 128
_NSC = 2
_NTILE = 16
_RA = 157
_RB = 79
_SSEG = _B * _NP // _NTILE
_DSEG = _NP // _NTILE
_NSELF = (_N + 127) // 128
_NBPAD = _NSELF * 128


def _sc_build(edge_flat, batch):
    mesh = plsc.VectorSubcoreMesh(core_axis_name="c", subcore_axis_name="s")

    @functools.partial(
        pl.kernel,
        mesh=mesh,
        compiler_params=pltpu.CompilerParams(needs_layout_passes=False),
        out_type=[
            jax.ShapeDtypeStruct((_NSC * _B * _NP,), jnp.float32),
            jax.ShapeDtypeStruct((_B,), jnp.float32),
        ],
        scratch_types=[
            pltpu.VMEM((_RA * 128,), jnp.int32),
            pltpu.VMEM((_RB * 128,), jnp.int32),
            pltpu.VMEM((_RB * 128,), jnp.int32),
            pltpu.VMEM((_NBPAD,), jnp.int32),
            pltpu.VMEM((_NP,), jnp.float32),
            pltpu.VMEM((_NP,), jnp.float32),
            pltpu.VMEM((4 * _DSEG,), jnp.float32),
            pltpu.VMEM((_B,), jnp.float32),
            pltpu.VMEM((_B,), jnp.int32),
        ] + [pltpu.VMEM((128,), jnp.int32) for _ in range(8)]
          + [pltpu.VMEM((128,), jnp.float32) for _ in range(8)]
          + [pltpu.SemaphoreType.DMA for _ in range(10)]
          + [
            pltpu.VMEM_SHARED((_NP,), jnp.float32),
            pltpu.VMEM_SHARED((_NTILE, _NP), jnp.float32),
            pltpu.VMEM_SHARED((_B,), jnp.float32),
            pltpu.VMEM_SHARED((_B * _NP,), jnp.float32),
        ],
    )
    def sc(edge_h, batch_h, s_out, cnt_out,
           dstA, srcB, dstB, batch_v, dis_v, histv, tmpred, cntv, idc,
           ib0, ib1, ib2, ib3, ib4, ib5, ib6, ib7,
           vb0, vb1, vb2, vb3, vb4, vb5, vb6, vb7,
           sm0, sm1, sm2, sm3, sm4, sm5, sm6, sm7, sm8, sm9,
           dis_sh, hist_sh, cnt_sh, s_sh):
        idx_bufs = (ib0, ib1, ib2, ib3, ib4, ib5, ib6, ib7)
        val_bufs = (vb0, vb1, vb2, vb3, vb4, vb5, vb6, vb7)
        sems = (sm0, sm1, sm2, sm3, sm6, sm7, sm8, sm9)
        c = lax.axis_index("c")
        s = lax.axis_index("s")
        w = s * _NSC + c

        base_a = (s * _ROWS) // _NTILE
        end_a = ((s + 1) * _ROWS) // _NTILE
        base_b = (w * _ROWS) // (_NSC * _NTILE)
        end_b = ((w + 1) * _ROWS) // (_NSC * _NTILE)

        pltpu.async_copy(
            edge_h.at[1, pl.ds(base_a * 128, _RA * 128)], dstA, sm4)
        pltpu.async_copy(batch_h, batch_v.at[pl.ds(0, _N)], sm4)
        pltpu.async_copy(
            edge_h.at[0, pl.ds(base_b * 128, _RB * 128)], srcB, sm5)
        pltpu.async_copy(
            edge_h.at[1, pl.ds(base_b * 128, _RB * 128)], dstB, sm5)

        iota16 = lax.iota(jnp.int32, 16)
        zf = jnp.zeros((16,), jnp.float32)
        zi = jnp.zeros((16,), jnp.int32)

        @functools.partial(plsc.parallel_loop, 0, _NP // 16, unroll=4)
        def _zero(i):
            dis_v[pl.ds(i * 16, 16)] = zf
            histv[pl.ds(i * 16, 16)] = zf
        for t in range((_NBPAD - _N) // 16):
            batch_v[pl.ds(_N + t * 16, 16)] = zi
        for t in range(_B // 16):
            cntv[pl.ds(t * 16, 16)] = zf
            idc[pl.ds(t * 16, 16)] = iota16 + 16 * t

        for t in range(_SSEG // _NP):
            pltpu.async_copy(dis_v, s_sh.at[pl.ds(s * _SSEG + t * _NP, _NP)],
                             sems[t])

        @pl.when(s == 0)
        def _():
            pltpu.sync_copy(dis_v.at[pl.ds(0, _B)], cnt_sh)

        pltpu.make_async_copy(
            edge_h.at[1, pl.ds(base_a * 128, _RA * 128)], dstA, sm4).wait()
        pltpu.make_async_copy(batch_h, batch_v.at[pl.ds(0, _N)], sm4).wait()

        ones16 = jnp.ones((16,), jnp.float32)

        @functools.partial(plsc.parallel_loop, 0, 156 * 8, unroll=4)
        def _phase_a(i):
            dd = dstA[pl.ds(i * 16, 16)]
            plsc.addupdate_scatter(histv, [dd], ones16)

        @pl.when(end_a - base_a > 156)
        def _():
            for k in range(8):
                dd = dstA[pl.ds(156 * 128 + k * 16, 16)]
                plsc.addupdate_scatter(histv, [dd], ones16)

        pltpu.sync_copy(histv, hist_sh.at[s])
        plsc.subcore_barrier()
        for p in range(_NTILE // 4):
            for r4 in range(4):
                r = p * 4 + r4
                pltpu.async_copy(
                    hist_sh.at[r, pl.ds(s * _DSEG, _DSEG)],
                    tmpred.at[pl.ds(r4 * _DSEG, _DSEG)], sm4)
            for r4 in range(4):
                r = p * 4 + r4
                pltpu.make_async_copy(
                    hist_sh.at[r, pl.ds(s * _DSEG, _DSEG)],
                    tmpred.at[pl.ds(r4 * _DSEG, _DSEG)], sm4).wait()

            @functools.partial(
                plsc.parallel_loop, 0, _DSEG // 16, unroll=4)
            def _reduce(i, p=p):
                acc = tmpred[pl.ds(i * 16, 16)]
                for r4 in range(1, 4):
                    acc = acc + tmpred[pl.ds(r4 * _DSEG + i * 16, 16)]
                if p > 0:
                    acc = acc + histv[pl.ds(i * 16, 16)]
                histv[pl.ds(i * 16, 16)] = acc

        @functools.partial(plsc.parallel_loop, 0, _DSEG // 16, unroll=4)
        def _mkdis(i):
            d = histv[pl.ds(i * 16, 16)] + 1.0
            seed = plsc.bitcast(
                jnp.int32(0x5F3759DF) - lax.shift_right_logical(
                    plsc.bitcast(d, jnp.int32), 1),
                jnp.float32)
            hd = 0.5 * d
            y = seed
            for _ in range(3):
                y = y * (1.5 - hd * y * y)
            histv[pl.ds(i * 16, 16)] = y
        pltpu.sync_copy(histv.at[pl.ds(0, _DSEG)],
                        dis_sh.at[pl.ds(s * _DSEG, _DSEG)])

        for t in range(_SSEG // _NP):
            pltpu.make_async_copy(
                dis_v, s_sh.at[pl.ds(s * _SSEG + t * _NP, _NP)],
                sems[t]).wait()

        plsc.subcore_barrier()
        pltpu.sync_copy(dis_sh, dis_v)

        for j in range(3):
            cch = w + _NSC * _NTILE * j

            @pl.when(cch < _NSELF)
            def _(cch=cch):
                for k in range(8):
                    off = cch * 128 + k * 16
                    nn = off + iota16
                    bb = batch_v[pl.ds(off, 16)]
                    fs = dis_v[pl.ds(off, 16)]
                    idx_bufs[j][pl.ds(k * 16, 16)] = bb * _NP + nn
                    val_bufs[j][pl.ds(k * 16, 16)] = jnp.where(
                        nn < _N, fs * fs, 0.0)
                pltpu.async_copy(val_bufs[j], s_sh.at[idx_bufs[j]],
                                 sems[j], add=True)

        for j in range(3):
            cch = w + _NSC * _NTILE * j

            @pl.when(cch < _NSELF)
            def _(j=j, cch=cch):
                pltpu.make_async_copy(
                    val_bufs[j], s_sh.at[idx_bufs[j]], sems[j]).wait()

        for j in range(5):
            cch = s + _NTILE * j

            @pl.when(cch < _NSELF)
            def _(cch=cch):
                for k in range(8):
                    off = cch * 128 + k * 16
                    bb = batch_v[pl.ds(off, 16)]
                    ok = (off + iota16) < _N
                    plsc.addupdate_scatter(
                        cntv, [bb], jnp.where(ok, 1.0, 0.0))
        pltpu.sync_copy(cntv, cnt_sh.at[idc], add=True)

        pltpu.make_async_copy(
            edge_h.at[0, pl.ds(base_b * 128, _RB * 128)], srcB, sm5).wait()
        pltpu.make_async_copy(
            edge_h.at[1, pl.ds(base_b * 128, _RB * 128)], dstB, sm5).wait()

        n_b = end_b - base_b

        def _pb_chunk(j, b):
            for k in range(8):
                o = j * 128 + k * 16
                dd = dstB[pl.ds(o, 16)]
                ss = srcB[pl.ds(o, 16)]
                bb = plsc.load_gather(batch_v, [dd])
                fd = plsc.load_gather(dis_v, [dd])
                fs = plsc.load_gather(dis_v, [ss])
                idx_bufs[b][pl.ds(k * 16, 16)] = bb * _NP + ss
                val_bufs[b][pl.ds(k * 16, 16)] = fd * fs
            pltpu.async_copy(val_bufs[b], s_sh.at[idx_bufs[b]],
                             sems[b], add=True)

        def _pb_wait(b):
            pltpu.make_async_copy(
                val_bufs[b], s_sh.at[idx_bufs[b]], sems[b]).wait()

        for b in range(8):
            _pb_chunk(b, b)

        def _phase_b(g, carry):
            for b in range(8):
                j = g * 8 + b
                _pb_wait(b)
                _pb_chunk(j, b)
            return carry
        lax.fori_loop(1, 9, _phase_b, 0)

        for j in range(72, _RB):
            b = j % 8

            @pl.when(j < n_b)
            def _(j=j, b=b):
                _pb_wait(b)
                _pb_chunk(j, b)

        for b in range(8):
            _pb_wait(b)

        plsc.subcore_barrier()

        for t in range(_SSEG // _NP):
            pltpu.async_copy(
                s_sh.at[pl.ds(s * _SSEG + t * _NP, _NP)],
                s_out.at[pl.ds(c * (_B * _NP) + s * _SSEG + t * _NP, _NP)],
                sems[t])
        for t in range(_SSEG // _NP):
            pltpu.make_async_copy(
                s_sh.at[pl.ds(s * _SSEG + t * _NP, _NP)],
                s_out.at[pl.ds(c * (_B * _NP) + s * _SSEG + t * _NP, _NP)],
                sems[t]).wait()

        @pl.when(jnp.logical_and(c == 0, s == 0))
        def _():
            pltpu.sync_copy(cnt_sh, cnt_out)

    return sc(edge_flat, batch)


_KBLK = 2048
_KSTEPS = _NP // _KBLK


def _tc_dense(s_slabs, x_p, cnt2, wc, bc, w1, b1, w2, b2, w3, b3):
    def body(s_ref, x_ref, cnt_ref, wc_ref, bc_ref, w1_ref, b1_ref,
             w2_ref, b2_ref, w3_ref, b3_ref, o_ref, acc):
        k = pl.program_id(0)

        @pl.when(k == 0)
        def _():
            acc[...] = jnp.zeros_like(acc)

        sv = s_ref[...]
        acc[...] += jnp.dot(sv[0] + sv[1], x_ref[...],
                            preferred_element_type=jnp.float32,
                            precision=lax.Precision.HIGHEST)

        @pl.when(k == _KSTEPS - 1)
        def _():
            dot = functools.partial(jnp.dot,
                                    preferred_element_type=jnp.float32,
                                    precision=lax.Precision.HIGHEST)
            pooled = dot(acc[...], wc_ref[...]) + cnt_ref[...] * bc_ref[...]
            h1 = jnp.maximum(dot(pooled, w1_ref[...]) + b1_ref[...], 0.0)
            h2 = jnp.maximum(dot(h1, w2_ref[...]) + b2_ref[...], 0.0)
            lg = dot(h2, w3_ref[...]) + b3_ref[...]
            mx = jnp.max(lg, axis=1, keepdims=True)
            ex = jnp.exp(lg - mx)
            o_ref[...] = ex / jnp.sum(ex, axis=1, keepdims=True)

    full = lambda a: pl.BlockSpec(a.shape, lambda k: (0,) * a.ndim)
    return pl.pallas_call(
        body,
        grid=(_KSTEPS,),
        in_specs=[
            pl.BlockSpec((2, _B, _KBLK), lambda k: (0, 0, k)),
            pl.BlockSpec((_KBLK, 128), lambda k: (k, 0)),
            full(cnt2),
            full(wc), full(bc), full(w1), full(b1),
            full(w2), full(b2), full(w3), full(b3),
        ],
        out_specs=pl.BlockSpec((_B, 2), lambda k: (0, 0)),
        out_shape=jax.ShapeDtypeStruct((_B, 2), jnp.float32),
        scratch_shapes=[
            pltpu.VMEM((_B, 128), jnp.float32),
        ],
    )(s_slabs, x_p, cnt2, wc, bc, w1, b1, w2, b2, w3, b3)


def kernel(x, edge_index, batch, W_conv, b_conv, W1, b1, W2, b2, W3, b3):
    edge_i = jnp.asarray(edge_index, jnp.int32)
    batch_i = jnp.asarray(batch, jnp.int32)
    s_flat, cnt = _sc_build(edge_i, batch_i)
    s_slabs = s_flat.reshape(_NSC, _B, _NP)
    x_p = jnp.pad(x, ((0, _NP - _N), (0, 0)))
    return _tc_dense(s_slabs, x_p, cnt.reshape(_B, 1),
                     W_conv, b_conv.reshape(1, -1), W1, b1.reshape(1, -1),
                     W2, b2.reshape(1, -1), W3, b3.reshape(1, -1))
